# baseline (speedup 1.0000x reference)
import functools

import jax
import jax.numpy as jnp
from jax import lax
from jax.experimental import pallas as pl
from jax.experimental.pallas import tpu as pltpu
from jax.experimental.pallas import tpu_sc as plsc

EMB = 128
CTX = 200
H1 = 1024
OUT = 128

_NC, _NS = 2, 16
_ROWS_W = 16
_PCTX = 256

_HB = 128
_NH = (H1 // 2) // _HB


def _gather_body(left_hbm, right_hbm, table_hbm, out_hbm,
                 idxl_v, idxr_v, rows_v, sem):
    wid = lax.axis_index("s") * _NC + lax.axis_index("c")
    tower = wid // 16
    slot = wid % 16
    base = jnp.minimum(slot * _ROWS_W, CTX - _ROWS_W)
    pltpu.sync_copy(left_hbm.at[pl.ds(base, _ROWS_W)], idxl_v)
    pltpu.sync_copy(right_hbm.at[pl.ds(base, _ROWS_W)], idxr_v)
    idx = jnp.where(tower == 0, idxl_v[...], idxr_v[...])
    pltpu.async_copy(table_hbm.at[idx], rows_v, sem).wait()
    pltpu.sync_copy(rows_v, out_hbm.at[pl.ds(tower * _PCTX + base, _ROWS_W)])


@functools.cache
def _make_gather():
    return functools.partial(
        pl.kernel,
        mesh=plsc.VectorSubcoreMesh(core_axis_name="c", subcore_axis_name="s"),
        out_type=jax.ShapeDtypeStruct((2 * _PCTX, EMB), jnp.float32),
        scratch_types=[
            pltpu.VMEM((_ROWS_W,), jnp.int32),
            pltpu.VMEM((_ROWS_W,), jnp.int32),
            pltpu.VMEM((_ROWS_W, EMB), jnp.float32),
            pltpu.SemaphoreType.DMA,
        ],
    )(_gather_body)


_NSLAB = H1 // _HB
_NBUF = 4


def _mlp_body(rows_ref, w1_hbm, b1_ref, w2_ref, b2_ref,
              outl_ref, outr_ref, h_ref, x_ref, wbuf_ref, sems):
    for b in range(_NBUF - 1):
        pltpu.make_async_copy(
            w1_hbm.at[pl.ds(b * _HB, _HB)], wbuf_ref.at[b], sems.at[b]).start()
    x_ref[...] = rows_ref[:, :CTX, :].reshape(2, CTX * EMB)
    for k in range(_NSLAB):
        b = k % _NBUF
        pltpu.make_async_copy(
            w1_hbm.at[pl.ds(k * _HB, _HB)], wbuf_ref.at[b], sems.at[b]).wait()
        if k + _NBUF - 1 < _NSLAB:
            nk = k + _NBUF - 1
            nb = nk % _NBUF
            pltpu.make_async_copy(
                w1_hbm.at[pl.ds(nk * _HB, _HB)], wbuf_ref.at[nb],
                sems.at[nb]).start()
        xa = lax.dot_general(
            x_ref[...], wbuf_ref[b],
            (((1,), (1,)), ((), ())),
            preferred_element_type=jnp.float32)
        h_ref[:, k * _HB:(k + 1) * _HB] = jnp.maximum(
            xa + b1_ref[:, k * _HB:(k + 1) * _HB], 0.0)
    o = lax.dot_general(
        h_ref[...], w2_ref[...],
        (((1,), (1,)), ((), ())),
        preferred_element_type=jnp.float32)
    o = jnp.maximum(o + b2_ref[...], 0.0)
    outl_ref[...] = o[0:1, :]
    outr_ref[...] = o[1:2, :]


def _mlp(rows, W1, b1, W2, b2):
    return pl.pallas_call(
        _mlp_body,
        in_specs=[
            pl.BlockSpec((2, _PCTX, EMB), lambda: (0, 0, 0)),
            pl.BlockSpec(memory_space=pl.ANY),
            pl.BlockSpec((1, H1), lambda: (0, 0)),
            pl.BlockSpec((OUT, H1), lambda: (0, 0)),
            pl.BlockSpec((1, OUT), lambda: (0, 0)),
        ],
        out_specs=[
            pl.BlockSpec((1, OUT), lambda: (0, 0)),
            pl.BlockSpec((1, OUT), lambda: (0, 0)),
        ],
        out_shape=[
            jax.ShapeDtypeStruct((1, OUT), jnp.float32),
            jax.ShapeDtypeStruct((1, OUT), jnp.float32),
        ],
        scratch_shapes=[
            pltpu.VMEM((2, H1), jnp.float32),
            pltpu.VMEM((2, CTX * EMB), jnp.float32),
            pltpu.VMEM((_NBUF, _HB, CTX * EMB), jnp.float32),
            pltpu.SemaphoreType.DMA((_NBUF,)),
        ],
    )(rows, W1, b1, W2, b2)


def kernel(inputs_left, inputs_right, emb, W1, b1, W2, b2):
    rows = _make_gather()(
        inputs_left.astype(jnp.int32), inputs_right.astype(jnp.int32), emb)
    out_l, out_r = _mlp(rows.reshape(2, _PCTX, EMB), W1,
                        b1.reshape(1, H1), W2, b2.reshape(1, OUT))
    return (out_l, out_r)


# speedup vs baseline: 1.0401x; 1.0401x over previous
import functools

import jax
import jax.numpy as jnp
from jax import lax
from jax.experimental import pallas as pl
from jax.experimental.pallas import tpu as pltpu
from jax.experimental.pallas import tpu_sc as plsc

EMB = 128
CTX = 200
H1 = 1024
OUT = 128

_NC, _NS = 2, 16
_ROWS_W = 16
_PCTX = 256

_HB = 128
_NH = (H1 // 2) // _HB


def _gather_body(left_hbm, right_hbm, table_hbm, out_hbm,
                 idxl_v, idxr_v, rows_v, sem, sem2):
    wid = lax.axis_index("s") * _NC + lax.axis_index("c")
    tower = wid // 16
    slot = wid % 16
    base = jnp.minimum(slot * _ROWS_W, CTX - _ROWS_W)
    cl = pltpu.make_async_copy(left_hbm.at[pl.ds(base, _ROWS_W)], idxl_v, sem)
    cr = pltpu.make_async_copy(right_hbm.at[pl.ds(base, _ROWS_W)], idxr_v,
                               sem2)
    cl.start()
    cr.start()
    cl.wait()
    cr.wait()
    idx = jnp.where(tower == 0, idxl_v[...], idxr_v[...])
    pltpu.async_copy(table_hbm.at[idx], rows_v, sem).wait()
    pltpu.sync_copy(rows_v, out_hbm.at[pl.ds(tower * _PCTX + base, _ROWS_W)])


@functools.cache
def _make_gather():
    return functools.partial(
        pl.kernel,
        mesh=plsc.VectorSubcoreMesh(core_axis_name="c", subcore_axis_name="s"),
        out_type=jax.ShapeDtypeStruct((2 * _PCTX, EMB), jnp.float32),
        scratch_types=[
            pltpu.VMEM((_ROWS_W,), jnp.int32),
            pltpu.VMEM((_ROWS_W,), jnp.int32),
            pltpu.VMEM((_ROWS_W, EMB), jnp.float32),
            pltpu.SemaphoreType.DMA,
            pltpu.SemaphoreType.DMA,
        ],
    )(_gather_body)


def _mlp_body(rows_ref, w1a_ref, w1b_ref, b1_ref, w2_ref, b2_ref,
              outl_ref, outr_ref, h_ref, x_ref):
    k = pl.program_id(0)

    @pl.when(k == 0)
    def _():
        x_ref[...] = rows_ref[:, :CTX, :].reshape(2, CTX * EMB)

    xa = lax.dot_general(
        x_ref[...], w1a_ref[...],
        (((1,), (1,)), ((), ())),
        preferred_element_type=jnp.float32)
    h_ref[:, pl.ds(2 * k * _HB, _HB)] = jnp.maximum(
        xa + b1_ref[:, pl.ds(2 * k * _HB, _HB)], 0.0)
    xb = lax.dot_general(
        x_ref[...], w1b_ref[...],
        (((1,), (1,)), ((), ())),
        preferred_element_type=jnp.float32)
    h_ref[:, pl.ds((2 * k + 1) * _HB, _HB)] = jnp.maximum(
        xb + b1_ref[:, pl.ds((2 * k + 1) * _HB, _HB)], 0.0)

    @pl.when(k == _NH - 1)
    def _():
        o = lax.dot_general(
            h_ref[...], w2_ref[...],
            (((1,), (1,)), ((), ())),
            preferred_element_type=jnp.float32)
        o = jnp.maximum(o + b2_ref[...], 0.0)
        outl_ref[...] = o[0:1, :]
        outr_ref[...] = o[1:2, :]


def _mlp(rows, W1, b1, W2, b2):
    return pl.pallas_call(
        _mlp_body,
        grid=(_NH,),
        in_specs=[
            pl.BlockSpec((2, _PCTX, EMB), lambda k: (0, 0, 0)),
            pl.BlockSpec((_HB, CTX * EMB), lambda k: (2 * k, 0)),
            pl.BlockSpec((_HB, CTX * EMB), lambda k: (2 * k + 1, 0)),
            pl.BlockSpec((1, H1), lambda k: (0, 0)),
            pl.BlockSpec((OUT, H1), lambda k: (0, 0)),
            pl.BlockSpec((1, OUT), lambda k: (0, 0)),
        ],
        out_specs=[
            pl.BlockSpec((1, OUT), lambda k: (0, 0)),
            pl.BlockSpec((1, OUT), lambda k: (0, 0)),
        ],
        out_shape=[
            jax.ShapeDtypeStruct((1, OUT), jnp.float32),
            jax.ShapeDtypeStruct((1, OUT), jnp.float32),
        ],
        scratch_shapes=[
            pltpu.VMEM((2, H1), jnp.float32),
            pltpu.VMEM((2, CTX * EMB), jnp.float32),
        ],
    )(rows, W1, W1, b1, W2, b2)


def kernel(inputs_left, inputs_right, emb, W1, b1, W2, b2):
    rows = _make_gather()(
        inputs_left.astype(jnp.int32), inputs_right.astype(jnp.int32), emb)
    out_l, out_r = _mlp(rows.reshape(2, _PCTX, EMB), W1,
                        b1.reshape(1, H1), W2, b2.reshape(1, OUT))
    return (out_l, out_r)
